# baseline (speedup 1.0000x reference)
import functools

import jax
import jax.numpy as jnp
from jax.experimental import pallas as pl
from jax.experimental.pallas import tpu as pltpu
from jax.experimental.pallas import tpu_sc as plsc

_MAX_MEL = 2048
_HH = 128
_OUT_CHUNK = 128


def _sc_expand(x4, duration):
    b_dim, s_dim, _, _ = x4.shape
    m_dim = _MAX_MEL
    n_chunks = m_dim // _OUT_CHUNK
    mesh = plsc.VectorSubcoreMesh(
        core_axis_name="core", subcore_axis_name="subcore")

    @functools.partial(
        pl.kernel,
        out_type=[
            jax.ShapeDtypeStruct((b_dim, m_dim, 2, _HH), jnp.float32),
            jax.ShapeDtypeStruct((b_dim, 16), jnp.int32),
        ],
        mesh=mesh,
        scratch_types=[
            pltpu.VMEM((s_dim, _HH), jnp.float32),
            pltpu.VMEM((s_dim,), jnp.int32),
            pltpu.VMEM((m_dim,), jnp.int32),
            pltpu.VMEM((16,), jnp.int32),
            pltpu.VMEM((2, _OUT_CHUNK, _HH), jnp.float32),
            pltpu.SemaphoreType.DMA,
            pltpu.SemaphoreType.DMA,
            pltpu.SemaphoreType.DMA,
            pltpu.SemaphoreType.DMA,
        ])
    def expand_kernel(x_hbm, d_hbm, o_hbm, l_hbm,
                      xl_v, dur_v, idx_v, mel_v, obuf, ssem, lsem, os0, os1):
        osems = (os0, os1)
        wid = (jax.lax.axis_index("subcore") * 2
               + jax.lax.axis_index("core"))
        b = wid // 2
        h = wid % 2
        stage = pltpu.async_copy(x_hbm.at[b, :, h], xl_v, ssem)
        pltpu.sync_copy(d_hbm.at[b], dur_v)

        lanes = jax.lax.broadcasted_iota(jnp.int32, (16,), 0)
        zeros = jnp.zeros((16,), jnp.int32)

        @pl.loop(0, m_dim // 16)
        def _(g):
            idx_v[pl.ds(g * 16, 16)] = zeros

        def expand_group(g, p):
            dvec = dur_v[pl.ds(g * 16, 16)]
            for k in range(16):
                d = dvec[k]
                svec = jnp.full((16,), g * 16 + k, jnp.int32)
                base = (p >> 4) << 4
                lo = p - base
                hi = lo + d
                cur0 = idx_v[pl.ds(base, 16)]
                m0 = (lanes >= lo) & (lanes < hi)
                idx_v[pl.ds(base, 16)] = jnp.where(m0, svec, cur0)
                cur1 = idx_v[pl.ds(base + 16, 16)]
                m1 = (lanes + 16) < hi
                idx_v[pl.ds(base + 16, 16)] = jnp.where(m1, svec, cur1)
                p = p + d
            return p

        total = jax.lax.fori_loop(0, s_dim // 16, expand_group, 0)

        @pl.when(h == 0)
        def _():
            mel_v[...] = jnp.broadcast_to(
                jnp.minimum(total, m_dim), (16,)).astype(jnp.int32)
            pltpu.async_copy(mel_v, l_hbm.at[b], lsem).wait()

        stage.wait()

        def out_slice(c):
            return o_hbm.at[b, pl.ds(c * _OUT_CHUNK, _OUT_CHUNK), h]

        def fill_chunk(c, buf):
            dst = obuf.at[buf]

            @plsc.parallel_loop(0, _OUT_CHUNK // 16, unroll=2)
            def _(g):
                li_vec = idx_v[pl.ds(c * _OUT_CHUNK + g * 16, 16)]
                for k in range(16):
                    li = li_vec[k]
                    src_row = xl_v.at[li]
                    dst_row = dst.at[g * 16 + k]
                    for j in range(_HH // 16):
                        dst_row[pl.ds(16 * j, 16)] = src_row[pl.ds(16 * j, 16)]

        @pl.loop(0, n_chunks // 2)
        def _(i):
            c0 = i * 2
            for buf in range(2):
                c = c0 + buf

                @pl.when(i > 0)
                def _():
                    pltpu.make_async_copy(
                        obuf.at[buf], out_slice(c - 2), osems[buf]).wait()

                fill_chunk(c, buf)
                pltpu.async_copy(obuf.at[buf], out_slice(c), osems[buf])

        for buf in range(2):
            pltpu.make_async_copy(
                obuf.at[buf], out_slice(n_chunks - 2 + buf),
                osems[buf]).wait()

    return expand_kernel(x4, duration)


def kernel(x, duration, max_len):
    b, s, h = x.shape
    out, mel = _sc_expand(x.reshape(b, s, 2, _HH), duration)
    return out.reshape(b, _MAX_MEL, h), mel[:, 0]
